# baseline (speedup 1.0000x reference)
import functools

import jax
import jax.numpy as jnp
from jax import lax
from jax.experimental import pallas as pl
from jax.experimental.pallas import tpu as pltpu
from jax.experimental.pallas import tpu_sc as plsc

N_NODES = 10000
D = 128
NC = 2
NS = 16
CHUNK = 128
NBUF = 2
N_PAD = 10240
ROWS_PER_TILE = N_PAD // NS
ZB = 128
G = 16
F0 = 0.7
BLK = 256

def _mesh():
    return plsc.VectorSubcoreMesh(core_axis_name="c", subcore_axis_name="s",
                                  num_cores=NC, num_subcores=NS)


def _sc_agg_body(cpt0, cpt1, x_hbm, src_hbm, dst_hbm, out_hbm, src_v, dst_v,
                 *refs):
    rows = refs[:NBUF]
    acc_sh = refs[NBUF]
    gsem = refs[NBUF + 1:2 * NBUF + 1]
    ssem = refs[2 * NBUF + 1:3 * NBUF + 1]
    cid = lax.axis_index("c")
    sid = lax.axis_index("s")
    n_stages = jnp.where(cid == 0, cpt0 // G, cpt1 // G)

    zero = jnp.zeros((16,), jnp.float32)

    def zrow(i, _):
        r = i // (D // 16)
        k = i % (D // 16)
        rows[0][r, pl.ds(k * 16, 16)] = zero
        return 0

    lax.fori_loop(0, CHUNK * (D // 16), zrow, 0)
    base = sid * ROWS_PER_TILE
    r = 0
    while r < ROWS_PER_TILE:
        n = min(CHUNK, ROWS_PER_TILE - r)
        pltpu.sync_copy(rows[0].at[pl.ds(0, n)],
                        acc_sh.at[pl.ds(base + r, n)])
        r += n
    plsc.subcore_barrier()

    def stage_body(g, _):
        pltpu.sync_copy(src_hbm.at[cid, sid, pl.ds(g * G, G)], src_v)
        pltpu.sync_copy(dst_hbm.at[cid, sid, pl.ds(g * G, G)], dst_v)

        gat = [None] * NBUF
        scat = [None] * NBUF
        for c in range(NBUF - 1):
            gat[c] = pltpu.async_copy(x_hbm.at[src_v.at[c]], rows[c], gsem[c])
        for c in range(G):
            b = c % NBUF
            gat[b].wait()
            n = c + NBUF - 1
            if n < G:
                bn = n % NBUF
                if scat[bn] is not None:
                    scat[bn].wait()
                gat[bn] = pltpu.async_copy(
                    x_hbm.at[src_v.at[n]], rows[bn], gsem[bn])
            scat[b] = pltpu.async_copy(
                rows[b], acc_sh.at[dst_v.at[c]], ssem[b], add=True)
        for c in range(G - NBUF, G):
            scat[c % NBUF].wait()
        return 0

    lax.fori_loop(0, n_stages, stage_body, 0)
    plsc.subcore_barrier()

    for k in range(ROWS_PER_TILE // ZB):
        off = sid * ROWS_PER_TILE + k * ZB
        pltpu.sync_copy(acc_sh.at[pl.ds(off, ZB)],
                        out_hbm.at[cid, pl.ds(off, ZB)])


def _make_sc_agg(cpt0, cpt1):
    cpt = max(cpt0, cpt1)
    scratch = [
        pltpu.VMEM((G, CHUNK), jnp.int32),
        pltpu.VMEM((G, CHUNK), jnp.int32),
    ]
    scratch += [pltpu.VMEM((CHUNK, D), jnp.float32) for _ in range(NBUF)]
    scratch.append(pltpu.VMEM_SHARED((N_PAD, D), jnp.float32))
    scratch += [pltpu.SemaphoreType.DMA for _ in range(2 * NBUF)]
    return pl.kernel(
        functools.partial(_sc_agg_body, cpt0, cpt1),
        out_type=jax.ShapeDtypeStruct((NC, N_PAD, D), jnp.float32),
        mesh=_mesh(),
        scratch_types=tuple(scratch),
    )


def _sc_count_body(dst_hbm, cnt_hbm, idx_v, cnt_v):
    epw = dst_hbm.shape[2]
    cid = lax.axis_index("c")
    sid = lax.axis_index("s")
    pltpu.sync_copy(dst_hbm.at[cid, sid], idx_v)
    zero = jnp.zeros((16,), jnp.float32)

    def z(i, _):
        cnt_v[pl.ds(i * 16, 16)] = zero
        return 0

    lax.fori_loop(0, N_PAD // 16, z, 0)
    one = jnp.full((16,), 1.0, jnp.float32)

    def step(j, _):
        ix = idx_v[pl.ds(j * 16, 16)]
        plsc.addupdate_scatter(cnt_v, [ix], one)
        return 0

    lax.fori_loop(0, epw // 16, step, 0)
    pltpu.sync_copy(cnt_v, cnt_hbm.at[cid * NS + sid])


def _make_sc_count(epw):
    return pl.kernel(
        _sc_count_body,
        out_type=jax.ShapeDtypeStruct((NC * NS, N_PAD), jnp.float32),
        mesh=_mesh(),
        scratch_types=(
            pltpu.VMEM((epw,), jnp.int32),
            pltpu.VMEM((N_PAD,), jnp.float32),
        ),
        compiler_params=pltpu.CompilerParams(needs_layout_passes=False),
    )


def _tc_layer_body(relu, s_ref, cnt_ref, x_ref, wl_ref, wr_ref, b_ref, o_ref):
    s = s_ref[0] + s_ref[1]
    cnt = jnp.sum(cnt_ref[...], axis=1, keepdims=True)
    inv = 1.0 / jnp.maximum(cnt, 1.0)
    agg = s * inv
    h = (jnp.dot(agg, wl_ref[...], preferred_element_type=jnp.float32)
         + b_ref[0:1, :]
         + jnp.dot(x_ref[...], wr_ref[...], preferred_element_type=jnp.float32))
    o_ref[...] = jnp.maximum(h, 0.0) if relu else h


def _tc_layer(s_parts, cnt, x, wlt, wrt, b8, relu):
    return pl.pallas_call(
        functools.partial(_tc_layer_body, relu),
        grid=(N_PAD // BLK,),
        in_specs=[
            pl.BlockSpec((NC, BLK, D), lambda i: (0, i, 0)),
            pl.BlockSpec((BLK, NC * NS), lambda i: (i, 0)),
            pl.BlockSpec((BLK, D), lambda i: (i, 0)),
            pl.BlockSpec((D, D), lambda i: (0, 0)),
            pl.BlockSpec((D, D), lambda i: (0, 0)),
            pl.BlockSpec((8, D), lambda i: (0, 0)),
        ],
        out_specs=pl.BlockSpec((BLK, D), lambda i: (i, 0)),
        out_shape=jax.ShapeDtypeStruct((N_PAD, D), jnp.float32),
    )(s_parts, cnt, x, wlt, wrt, b8)


def kernel(x, edge_index, training, Wl0, Wr0, b0, Wl1, Wr1, b1,
           Wl2, Wr2, b2):
    src = edge_index[0].astype(jnp.int32)
    dst = edge_index[1].astype(jnp.int32)
    e = src.shape[0]
    total_chunks = -(-e // (NS * CHUNK))
    cpt0 = max(G, int(round(total_chunks * F0 / G)) * G)
    cpt1 = max(G, -(-max(total_chunks - cpt0, 0) // G) * G)
    e_pad = (cpt0 + cpt1) * NS * CHUNK
    pad = e_pad - e
    src_p = jnp.concatenate([src, jnp.full((pad,), N_NODES, jnp.int32)])
    dst_p = jnp.concatenate([dst, jnp.full((pad,), N_NODES, jnp.int32)])
    cm = max(cpt0, cpt1)

    def _split(a):
        p0 = a[:NS * cpt0 * CHUNK].reshape(NS, cpt0, CHUNK)
        p1 = a[NS * cpt0 * CHUNK:].reshape(NS, cpt1, CHUNK)
        p0 = jnp.pad(p0, ((0, 0), (0, cm - cpt0), (0, 0)))
        p1 = jnp.pad(p1, ((0, 0), (0, cm - cpt1), (0, 0)))
        return jnp.stack([p0, p1])

    srcr = _split(src_p)
    dstr = _split(dst_p)
    epw = e_pad // (NC * NS)
    dstf = dst_p.reshape(NC, NS, epw)

    x_pad = jnp.zeros((N_PAD, D), jnp.float32).at[:N_NODES].set(x)

    sc_agg = _make_sc_agg(cpt0, cpt1)
    sc_count = _make_sc_count(epw)

    wl0t, wr0t = Wl0.T, Wr0.T
    wl1t, wr1t = Wl1.T, Wr1.T
    wl2t, wr2t = Wl2.T, Wr2.T
    b0_8 = jnp.broadcast_to(b0[None, :], (8, D))
    b1_8 = jnp.broadcast_to(b1[None, :], (8, D))
    b2_8 = jnp.broadcast_to(b2[None, :], (8, D))

    cnt = sc_count(dstf).T
    s0 = sc_agg(x_pad, srcr, dstr)
    h0 = _tc_layer(s0, cnt, x_pad, wl0t, wr0t, b0_8, True)
    s1 = sc_agg(h0, srcr, dstr)
    h1 = _tc_layer(s1, cnt, h0, wl1t, wr1t, b1_8, True)
    s2 = sc_agg(h1, srcr, dstr)
    h2 = _tc_layer(s2, cnt, h1, wl2t, wr2t, b2_8, False)
    return h2[:N_NODES]


# speedup vs baseline: 1.0070x; 1.0070x over previous
import functools

import jax
import jax.numpy as jnp
from jax import lax
from jax.experimental import pallas as pl
from jax.experimental.pallas import tpu as pltpu
from jax.experimental.pallas import tpu_sc as plsc

N_NODES = 10000
D = 128
NC = 2
NS = 16
CHUNK = 128
NBUF = 2
N_PAD = 10240
ROWS_PER_TILE = N_PAD // NS
ZB = 128
G = 16
F0 = 0.8
BLK = 256

def _mesh():
    return plsc.VectorSubcoreMesh(core_axis_name="c", subcore_axis_name="s",
                                  num_cores=NC, num_subcores=NS)


def _sc_agg_body(cpt0, cpt1, x_hbm, src_hbm, dst_hbm, out_hbm, src_v, dst_v,
                 *refs):
    rows = refs[:NBUF]
    acc_sh = refs[NBUF]
    gsem = refs[NBUF + 1:2 * NBUF + 1]
    ssem = refs[2 * NBUF + 1:3 * NBUF + 1]
    cid = lax.axis_index("c")
    sid = lax.axis_index("s")
    n_stages = jnp.where(cid == 0, cpt0 // G, cpt1 // G)

    zero = jnp.zeros((16,), jnp.float32)

    def zrow(i, _):
        r = i // (D // 16)
        k = i % (D // 16)
        rows[0][r, pl.ds(k * 16, 16)] = zero
        return 0

    lax.fori_loop(0, CHUNK * (D // 16), zrow, 0)
    base = sid * ROWS_PER_TILE
    r = 0
    while r < ROWS_PER_TILE:
        n = min(CHUNK, ROWS_PER_TILE - r)
        pltpu.sync_copy(rows[0].at[pl.ds(0, n)],
                        acc_sh.at[pl.ds(base + r, n)])
        r += n
    plsc.subcore_barrier()

    def stage_body(g, _):
        pltpu.sync_copy(src_hbm.at[cid, sid, pl.ds(g * G, G)], src_v)
        pltpu.sync_copy(dst_hbm.at[cid, sid, pl.ds(g * G, G)], dst_v)

        gat = [None] * NBUF
        scat = [None] * NBUF
        for c in range(NBUF - 1):
            gat[c] = pltpu.async_copy(x_hbm.at[src_v.at[c]], rows[c], gsem[c])
        for c in range(G):
            b = c % NBUF
            gat[b].wait()
            n = c + NBUF - 1
            if n < G:
                bn = n % NBUF
                if scat[bn] is not None:
                    scat[bn].wait()
                gat[bn] = pltpu.async_copy(
                    x_hbm.at[src_v.at[n]], rows[bn], gsem[bn])
            scat[b] = pltpu.async_copy(
                rows[b], acc_sh.at[dst_v.at[c]], ssem[b], add=True)
        for c in range(G - NBUF, G):
            scat[c % NBUF].wait()
        return 0

    lax.fori_loop(0, n_stages, stage_body, 0)
    plsc.subcore_barrier()

    for k in range(ROWS_PER_TILE // ZB):
        off = sid * ROWS_PER_TILE + k * ZB
        pltpu.sync_copy(acc_sh.at[pl.ds(off, ZB)],
                        out_hbm.at[cid, pl.ds(off, ZB)])


def _make_sc_agg(cpt0, cpt1):
    cpt = max(cpt0, cpt1)
    scratch = [
        pltpu.VMEM((G, CHUNK), jnp.int32),
        pltpu.VMEM((G, CHUNK), jnp.int32),
    ]
    scratch += [pltpu.VMEM((CHUNK, D), jnp.float32) for _ in range(NBUF)]
    scratch.append(pltpu.VMEM_SHARED((N_PAD, D), jnp.float32))
    scratch += [pltpu.SemaphoreType.DMA for _ in range(2 * NBUF)]
    return pl.kernel(
        functools.partial(_sc_agg_body, cpt0, cpt1),
        out_type=jax.ShapeDtypeStruct((NC, N_PAD, D), jnp.float32),
        mesh=_mesh(),
        scratch_types=tuple(scratch),
    )


def _sc_count_body(dst_hbm, cnt_hbm, idx_v, cnt_v):
    epw = dst_hbm.shape[2]
    cid = lax.axis_index("c")
    sid = lax.axis_index("s")
    pltpu.sync_copy(dst_hbm.at[cid, sid], idx_v)
    zero = jnp.zeros((16,), jnp.float32)

    def z(i, _):
        cnt_v[pl.ds(i * 16, 16)] = zero
        return 0

    lax.fori_loop(0, N_PAD // 16, z, 0)
    one = jnp.full((16,), 1.0, jnp.float32)

    def step(j, _):
        ix = idx_v[pl.ds(j * 16, 16)]
        plsc.addupdate_scatter(cnt_v, [ix], one)
        return 0

    lax.fori_loop(0, epw // 16, step, 0)
    pltpu.sync_copy(cnt_v, cnt_hbm.at[cid * NS + sid])


def _make_sc_count(epw):
    return pl.kernel(
        _sc_count_body,
        out_type=jax.ShapeDtypeStruct((NC * NS, N_PAD), jnp.float32),
        mesh=_mesh(),
        scratch_types=(
            pltpu.VMEM((epw,), jnp.int32),
            pltpu.VMEM((N_PAD,), jnp.float32),
        ),
        compiler_params=pltpu.CompilerParams(needs_layout_passes=False),
    )


def _tc_layer_body(relu, s_ref, cnt_ref, x_ref, wl_ref, wr_ref, b_ref, o_ref):
    s = s_ref[0] + s_ref[1]
    cnt = jnp.sum(cnt_ref[...], axis=1, keepdims=True)
    inv = 1.0 / jnp.maximum(cnt, 1.0)
    agg = s * inv
    h = (jnp.dot(agg, wl_ref[...], preferred_element_type=jnp.float32)
         + b_ref[0:1, :]
         + jnp.dot(x_ref[...], wr_ref[...], preferred_element_type=jnp.float32))
    o_ref[...] = jnp.maximum(h, 0.0) if relu else h


def _tc_layer(s_parts, cnt, x, wlt, wrt, b8, relu):
    return pl.pallas_call(
        functools.partial(_tc_layer_body, relu),
        grid=(N_PAD // BLK,),
        in_specs=[
            pl.BlockSpec((NC, BLK, D), lambda i: (0, i, 0)),
            pl.BlockSpec((BLK, NC * NS), lambda i: (i, 0)),
            pl.BlockSpec((BLK, D), lambda i: (i, 0)),
            pl.BlockSpec((D, D), lambda i: (0, 0)),
            pl.BlockSpec((D, D), lambda i: (0, 0)),
            pl.BlockSpec((8, D), lambda i: (0, 0)),
        ],
        out_specs=pl.BlockSpec((BLK, D), lambda i: (i, 0)),
        out_shape=jax.ShapeDtypeStruct((N_PAD, D), jnp.float32),
    )(s_parts, cnt, x, wlt, wrt, b8)


def kernel(x, edge_index, training, Wl0, Wr0, b0, Wl1, Wr1, b1,
           Wl2, Wr2, b2):
    src = edge_index[0].astype(jnp.int32)
    dst = edge_index[1].astype(jnp.int32)
    e = src.shape[0]
    total_chunks = -(-e // (NS * CHUNK))
    cpt0 = max(G, int(round(total_chunks * F0 / G)) * G)
    cpt1 = max(G, -(-max(total_chunks - cpt0, 0) // G) * G)
    e_pad = (cpt0 + cpt1) * NS * CHUNK
    pad = e_pad - e
    src_p = jnp.concatenate([src, jnp.full((pad,), N_NODES, jnp.int32)])
    dst_p = jnp.concatenate([dst, jnp.full((pad,), N_NODES, jnp.int32)])
    cm = max(cpt0, cpt1)

    def _split(a):
        p0 = a[:NS * cpt0 * CHUNK].reshape(NS, cpt0, CHUNK)
        p1 = a[NS * cpt0 * CHUNK:].reshape(NS, cpt1, CHUNK)
        p0 = jnp.pad(p0, ((0, 0), (0, cm - cpt0), (0, 0)))
        p1 = jnp.pad(p1, ((0, 0), (0, cm - cpt1), (0, 0)))
        return jnp.stack([p0, p1])

    srcr = _split(src_p)
    dstr = _split(dst_p)
    epw = e_pad // (NC * NS)
    dstf = dst_p.reshape(NC, NS, epw)

    x_pad = jnp.zeros((N_PAD, D), jnp.float32).at[:N_NODES].set(x)

    sc_agg = _make_sc_agg(cpt0, cpt1)
    sc_count = _make_sc_count(epw)

    wl0t, wr0t = Wl0.T, Wr0.T
    wl1t, wr1t = Wl1.T, Wr1.T
    wl2t, wr2t = Wl2.T, Wr2.T
    b0_8 = jnp.broadcast_to(b0[None, :], (8, D))
    b1_8 = jnp.broadcast_to(b1[None, :], (8, D))
    b2_8 = jnp.broadcast_to(b2[None, :], (8, D))

    cnt = sc_count(dstf).T
    s0 = sc_agg(x_pad, srcr, dstr)
    h0 = _tc_layer(s0, cnt, x_pad, wl0t, wr0t, b0_8, True)
    s1 = sc_agg(h0, srcr, dstr)
    h1 = _tc_layer(s1, cnt, h0, wl1t, wr1t, b1_8, True)
    s2 = sc_agg(h1, srcr, dstr)
    h2 = _tc_layer(s2, cnt, h1, wl2t, wr2t, b2_8, False)
    return h2[:N_NODES]
